# baseline (speedup 1.0000x reference)
import functools

import jax
import jax.numpy as jnp
from jax import lax
from jax.experimental import pallas as pl
from jax.experimental.pallas import tpu as pltpu
from jax.experimental.pallas import tpu_sc as plsc

_HEAD = 0
_TAIL = 1
_BSZ, _SEQ, _HID = 128, 512, 1024
_LANES = 16

_NC = 2
_NS = 16
_NW = _NC * _NS
_ROWS_W = _BSZ // _NW
_GATHER_W = 2 * _ROWS_W


def _sc_gather_body(ids_hbm, table_hbm, out_hbm, ids_v, idx_v, rows_v, sem):
    wid = lax.axis_index("s") * _NC + lax.axis_index("c")
    rbase = wid * _ROWS_W

    pltpu.sync_copy(ids_hbm.at[pl.ds(rbase, _ROWS_W)], ids_v)

    idx_v[...] = jnp.zeros((_LANES,), jnp.int32)
    lane = jax.lax.broadcasted_iota(jnp.int32, (_LANES,), 0)
    for j in range(_ROWS_W):
        for k in range(_SEQ // _LANES):
            v = ids_v[j, pl.ds(k * _LANES, _LANES)]
            pos = lane + (k * _LANES)
            tgt = v + (2 * j)
            vals = (rbase + j) * _SEQ + pos
            plsc.store_scatter(idx_v, [tgt], vals, mask=v < 2)

    pltpu.async_copy(table_hbm.at[idx_v.at[pl.ds(0, _GATHER_W)]], rows_v, sem).wait()
    pltpu.sync_copy(
        rows_v.at[pl.ds(0, _GATHER_W)],
        out_hbm.at[pl.ds(wid * _GATHER_W, _GATHER_W)],
    )


_sc_gather = functools.partial(
    pl.kernel,
    out_type=jax.ShapeDtypeStruct((2 * _BSZ, _HID), jnp.float32),
    mesh=plsc.VectorSubcoreMesh(core_axis_name="c", subcore_axis_name="s"),
    scratch_types=[
        pltpu.VMEM((_ROWS_W, _SEQ), jnp.int32),
        pltpu.VMEM((_LANES,), jnp.int32),
        pltpu.VMEM((_GATHER_W, _HID), jnp.float32),
        pltpu.SemaphoreType.DMA,
    ],
    compiler_params=pltpu.CompilerParams(needs_layout_passes=False),
)(_sc_gather_body)


def _mm_body(ent_ref, w_ref, b_ref, out_ref):
    out_ref[...] = lax.dot_general(
        ent_ref[...], w_ref[...],
        dimension_numbers=(((1,), (1,)), ((), ())),
        preferred_element_type=jnp.float32,
    ) + b_ref[...]


def kernel(input_ids, last_hidden_state, W, b):
    table = last_hidden_state.reshape(_BSZ * _SEQ, _HID)
    slab = _sc_gather(input_ids, table)
    entity = slab.reshape(_BSZ, 2 * _HID)
    logits = pl.pallas_call(
        _mm_body,
        out_shape=jax.ShapeDtypeStruct((_BSZ, W.shape[0]), jnp.float32),
    )(entity, W, b.reshape(1, -1))
    return (logits, entity)


# speedup vs baseline: 1.0587x; 1.0587x over previous
import functools

import jax
import jax.numpy as jnp
from jax import lax
from jax.experimental import pallas as pl
from jax.experimental.pallas import tpu as pltpu
from jax.experimental.pallas import tpu_sc as plsc

_HEAD = 0
_TAIL = 1
_BSZ, _SEQ, _HID = 128, 512, 1024
_LANES = 16

_NC = 2
_NS = 16
_NW = _NC * _NS
_ROWS_W = _BSZ // _NW
_GATHER_W = 2 * _ROWS_W


def _sc_gather_body(ids_hbm, table_hbm, ent_hbm, ids_v, idx_v, rows_v, sem):
    wid = lax.axis_index("s") * _NC + lax.axis_index("c")
    rbase = wid * _ROWS_W

    pltpu.sync_copy(ids_hbm.at[pl.ds(rbase, _ROWS_W)], ids_v)

    idx_v[...] = jnp.zeros((_LANES,), jnp.int32)
    lane = jax.lax.broadcasted_iota(jnp.int32, (_LANES,), 0)
    base0 = rbase * _SEQ + lane
    for j in range(_ROWS_W):
        base_j = base0 + j * _SEQ
        for k in range(_SEQ // _LANES):
            v = ids_v[j, pl.ds(k * _LANES, _LANES)]
            vals = base_j + (k * _LANES)
            tgt = v * 4 + j
            plsc.store_scatter(idx_v, [tgt], vals, mask=v < 2)

    pltpu.async_copy(table_hbm.at[idx_v.at[pl.ds(0, _GATHER_W)]], rows_v, sem).wait()
    pltpu.sync_copy(
        rows_v.at[pl.ds(0, _ROWS_W)],
        ent_hbm.at[pl.ds(rbase, _ROWS_W), pl.ds(0, _HID)],
    )
    pltpu.sync_copy(
        rows_v.at[pl.ds(_ROWS_W, _ROWS_W)],
        ent_hbm.at[pl.ds(rbase, _ROWS_W), pl.ds(_HID, _HID)],
    )


_sc_gather = functools.partial(
    pl.kernel,
    out_type=jax.ShapeDtypeStruct((_BSZ, 2 * _HID), jnp.float32),
    mesh=plsc.VectorSubcoreMesh(core_axis_name="c", subcore_axis_name="s"),
    scratch_types=[
        pltpu.VMEM((_ROWS_W, _SEQ), jnp.int32),
        pltpu.VMEM((_LANES,), jnp.int32),
        pltpu.VMEM((_GATHER_W, _HID), jnp.float32),
        pltpu.SemaphoreType.DMA,
    ],
    compiler_params=pltpu.CompilerParams(needs_layout_passes=False),
)(_sc_gather_body)


def _mm_body(ent_ref, w_ref, b_ref, out_ref):
    out_ref[...] = lax.dot_general(
        ent_ref[...], w_ref[...],
        dimension_numbers=(((1,), (1,)), ((), ())),
        preferred_element_type=jnp.float32,
    ) + b_ref[...]


def kernel(input_ids, last_hidden_state, W, b):
    table = last_hidden_state.reshape(_BSZ * _SEQ, _HID)
    entity = _sc_gather(input_ids, table)
    logits = pl.pallas_call(
        _mm_body,
        out_shape=jax.ShapeDtypeStruct((_BSZ, W.shape[0]), jnp.float32),
    )(entity, W, b.reshape(1, -1))
    return (logits, entity)


# speedup vs baseline: 1.1376x; 1.0745x over previous
import functools

import jax
import jax.numpy as jnp
from jax import lax
from jax.experimental import pallas as pl
from jax.experimental.pallas import tpu as pltpu
from jax.experimental.pallas import tpu_sc as plsc

_HEAD = 0
_TAIL = 1
_BSZ, _SEQ, _HID = 128, 512, 1024
_LANES = 16

_NC = 2
_NS = 16
_NW = _NC * _NS
_ROWS_W = _BSZ // _NW
_GATHER_W = 2 * _ROWS_W


def _sc_gather_body(ids_hbm, table_hbm, ent_hbm, ids_v, idx_v, rows_v, sem):
    wid = lax.axis_index("s") * _NC + lax.axis_index("c")
    rbase = wid * _ROWS_W

    pltpu.sync_copy(ids_hbm.at[pl.ds(rbase, _ROWS_W)], ids_v)

    idx_v[...] = jnp.zeros((_LANES,), jnp.int32)
    lane = jax.lax.broadcasted_iota(jnp.int32, (_LANES,), 0)
    base0 = rbase * _SEQ + lane
    for j in range(_ROWS_W):
        base_j = base0 + j * _SEQ

        @plsc.parallel_loop(0, _SEQ, step=_LANES, unroll=4)
        def _scan(p, j=j, base_j=base_j):
            v = ids_v[j, pl.ds(p, _LANES)]
            plsc.store_scatter(idx_v, [v * 4 + j], base_j + p, mask=v < 2)

    pltpu.async_copy(table_hbm.at[idx_v.at[pl.ds(0, _GATHER_W)]], rows_v, sem).wait()
    pltpu.sync_copy(
        rows_v.at[pl.ds(0, _ROWS_W)],
        ent_hbm.at[pl.ds(rbase, _ROWS_W), pl.ds(0, _HID)],
    )
    pltpu.sync_copy(
        rows_v.at[pl.ds(_ROWS_W, _ROWS_W)],
        ent_hbm.at[pl.ds(rbase, _ROWS_W), pl.ds(_HID, _HID)],
    )


_sc_gather = functools.partial(
    pl.kernel,
    out_type=jax.ShapeDtypeStruct((_BSZ, 2 * _HID), jnp.float32),
    mesh=plsc.VectorSubcoreMesh(core_axis_name="c", subcore_axis_name="s"),
    scratch_types=[
        pltpu.VMEM((_ROWS_W, _SEQ), jnp.int32),
        pltpu.VMEM((_LANES,), jnp.int32),
        pltpu.VMEM((_GATHER_W, _HID), jnp.float32),
        pltpu.SemaphoreType.DMA,
    ],
    compiler_params=pltpu.CompilerParams(needs_layout_passes=False),
)(_sc_gather_body)


def _mm_body(ent_ref, w_ref, b_ref, out_ref):
    out_ref[...] = lax.dot_general(
        ent_ref[...], w_ref[...],
        dimension_numbers=(((1,), (1,)), ((), ())),
        preferred_element_type=jnp.float32,
    ) + b_ref[...]


def kernel(input_ids, last_hidden_state, W, b):
    table = last_hidden_state.reshape(_BSZ * _SEQ, _HID)
    entity = _sc_gather(input_ids, table)
    logits = pl.pallas_call(
        _mm_body,
        out_shape=jax.ShapeDtypeStruct((_BSZ, W.shape[0]), jnp.float32),
    )(entity, W, b.reshape(1, -1))
    return (logits, entity)
